# Initial kernel scaffold; baseline (speedup 1.0000x reference)
import jax
import jax.numpy as jnp
from jax.experimental import pallas as pl


def kernel(x, W_gate, W_f1, b_f1, W_f2):
    raise NotImplementedError("write your pallas kernel here")


# speedup vs baseline: 8.8490x; 8.8490x over previous
import numpy as np
import jax
import jax.numpy as jnp
from jax.experimental import pallas as pl
from jax.experimental.pallas import tpu as pltpu

B, C, H, W = 256, 768, 14, 14
E = 16
K = 2
FREQ_BINS = 8
FREQ_DIM = 64
NOISE_STD = 1.0 / E
HW = H * W
NF = 128
NB = 8


def _build_dft():
    rep_col = {}
    cols = []
    rep_of = np.zeros(HW, np.int32)
    for h in range(H):
        for w in range(W):
            pair = ((H - h) % H, (W - w) % W)
            rep = min((h, w), pair)
            if rep not in rep_col:
                rep_col[rep] = len(cols)
                cols.append(rep)
            rep_of[h * W + w] = rep_col[rep]
    ii, jj = np.meshgrid(np.arange(H), np.arange(W), indexing="ij")
    fi = ii.reshape(-1).astype(np.float64)
    fj = jj.reshape(-1).astype(np.float64)
    M = np.zeros((HW, 2 * NF), np.float32)
    for k, (h, w) in enumerate(cols):
        ang = 2.0 * np.pi * (h * fi + w * fj) / H
        M[:, k] = (np.cos(ang) / HW).astype(np.float32)
        M[:, NF + k] = (-np.sin(ang) / HW).astype(np.float32)
    onehot = np.zeros((NF, HW), np.float32)
    onehot[rep_of, np.arange(HW)] = 1.0
    return M, onehot


_DFT_M, _REP_ONEHOT = _build_dft()


def _bin_weights():
    y = jnp.arange(-(H // 2), H // 2)
    xx = jnp.arange(-(W // 2), W // 2)
    gy, gx = jnp.meshgrid(y, xx, indexing="ij")
    grid = jnp.stack([gy, gx], axis=-1).astype(jnp.float32)
    fd = jnp.linalg.norm(grid, axis=-1)
    edges = jnp.linspace(0.0, fd.max(), FREQ_BINS + 1)
    masks = [((fd >= edges[i]) & (fd < edges[i + 1])).reshape(HW)
             for i in range(FREQ_BINS)]
    mm = jnp.stack(masks, axis=-1).astype(jnp.float32)
    return jnp.asarray(_REP_ONEHOT) @ mm


def _energy_kernel(x_ref, m_ref, bw_ref, wgt_ref, img_ref, femb_ref):
    xb = x_ref[...]
    x2 = xb.reshape(NB * C, HW)
    y = jnp.dot(x2, m_ref[...], preferred_element_type=jnp.float32)
    re = y[:, :NF]
    im = y[:, NF:]
    mag = jnp.sqrt(re * re + im * im)
    en = jnp.dot(mag, bw_ref[...], preferred_element_type=jnp.float32)
    femb_ref[...] = en.reshape(NB, C, FREQ_BINS).sum(axis=1) * (1.0 / C)
    ones = jnp.full((HW,), 1.0 / HW, jnp.float32)
    pooled = jax.lax.dot_general(xb, ones, (((2,), (0,)), ((), ())))
    img_ref[...] = jnp.dot(pooled, wgt_ref[...],
                           preferred_element_type=jnp.float32)


def _tail_kernel(img_ref, femb_ref, wf1t_ref, bf1_ref, wf2t_ref, noise_ref,
                 gates_ref, idx_ref, vals_ref, aux_ref):
    img = img_ref[...]
    femb = femb_ref[...]
    h = jnp.maximum(
        jnp.dot(femb, wf1t_ref[...], preferred_element_type=jnp.float32)
        + bf1_ref[...], 0.0)
    logits = img + jnp.dot(h, wf2t_ref[...],
                           preferred_element_type=jnp.float32)

    m = jnp.max(logits, axis=-1, keepdims=True)
    ex = jnp.exp(logits - m)
    s = ex / jnp.sum(ex, axis=-1, keepdims=True)
    imp = jnp.sum(s, axis=0, keepdims=True)
    imp_mean = jnp.sum(imp, axis=-1, keepdims=True) * (1.0 / E)
    imp_var = jnp.sum((imp - imp_mean) ** 2, axis=-1,
                      keepdims=True) * (1.0 / (E - 1))
    loss_imp = imp_var / (imp_mean + 1e-8) ** 2

    io = jax.lax.broadcasted_iota(jnp.int32, (B, E), 1)
    m1 = jnp.max(logits, axis=-1, keepdims=True)
    i1 = jnp.min(jnp.where(logits == m1, io, E), axis=-1, keepdims=True)
    lmask = jnp.where(io == i1, -jnp.inf, logits)
    thr = jnp.max(lmask, axis=-1, keepdims=True)
    z = (thr - logits) * (E * 0.7071067811865476)
    p = 0.5 - 0.5 * jax.lax.erf(z)
    pm = jnp.sum(p, axis=0, keepdims=True) * (1.0 / B)
    pmm = jnp.sum(pm, axis=-1, keepdims=True) * (1.0 / E)
    pvar = jnp.sum((pm - pmm) ** 2, axis=-1,
                   keepdims=True) * (1.0 / (E - 1))
    loss_load = pvar / (pmm + 1e-8) ** 2
    aux_ref[...] = 0.5 * loss_imp + 0.5 * loss_load

    nl = logits + noise_ref[...]
    nm = jnp.max(nl, axis=-1, keepdims=True)
    nex = jnp.exp(nl - nm)
    sc = nex / jnp.sum(nex, axis=-1, keepdims=True)
    v1 = jnp.max(sc, axis=-1, keepdims=True)
    j1 = jnp.min(jnp.where(sc == v1, io, E), axis=-1, keepdims=True)
    sc2 = jnp.where(io == j1, -1.0, sc)
    v2 = jnp.max(sc2, axis=-1, keepdims=True)
    j2 = jnp.min(jnp.where(sc2 == v2, io, E), axis=-1, keepdims=True)
    gates_ref[...] = jnp.where((io == j1) | (io == j2), sc, 0.0)
    idx_ref[...] = jnp.concatenate([j1, j2], axis=-1)
    vals_ref[...] = jnp.concatenate([v1, v2], axis=-1)


def kernel(x, W_gate, W_f1, b_f1, W_f2):
    x3 = x.reshape(B, C, HW)
    m_dev = jnp.asarray(_DFT_M)
    bw_dev = _bin_weights()
    wgt = W_gate.T

    img_logits, femb = pl.pallas_call(
        _energy_kernel,
        grid=(B // NB,),
        in_specs=[
            pl.BlockSpec((NB, C, HW), lambda i: (i, 0, 0)),
            pl.BlockSpec((HW, 2 * NF), lambda i: (0, 0)),
            pl.BlockSpec((NF, FREQ_BINS), lambda i: (0, 0)),
            pl.BlockSpec((C, E), lambda i: (0, 0)),
        ],
        out_specs=[
            pl.BlockSpec((NB, E), lambda i: (i, 0)),
            pl.BlockSpec((NB, FREQ_BINS), lambda i: (i, 0)),
        ],
        out_shape=[
            jax.ShapeDtypeStruct((B, E), jnp.float32),
            jax.ShapeDtypeStruct((B, FREQ_BINS), jnp.float32),
        ],
        compiler_params=pltpu.CompilerParams(
            dimension_semantics=("arbitrary",)),
    )(x3, m_dev, bw_dev, wgt)

    noise = jax.random.normal(jax.random.key(42), (B, E),
                              dtype=jnp.float32) * NOISE_STD
    gates, idx, vals, aux = pl.pallas_call(
        _tail_kernel,
        out_shape=[
            jax.ShapeDtypeStruct((B, E), jnp.float32),
            jax.ShapeDtypeStruct((B, K), jnp.int32),
            jax.ShapeDtypeStruct((B, K), jnp.float32),
            jax.ShapeDtypeStruct((1, 1), jnp.float32),
        ],
    )(img_logits, femb, W_f1.T, b_f1.reshape(1, FREQ_DIM), W_f2.T, noise)

    return gates, idx, vals, aux[0, 0]
